# Initial kernel scaffold; baseline (speedup 1.0000x reference)
import jax
import jax.numpy as jnp
from jax.experimental import pallas as pl


def kernel(x, W, I):
    raise NotImplementedError("write your pallas kernel here")


# speedup vs baseline: 9.4532x; 9.4532x over previous
import jax
import jax.numpy as jnp
from jax.experimental import pallas as pl
from jax.experimental.pallas import tpu as pltpu

_C, _G, _S, _L = 16, 1024, 32, 2
_B = 64
_M = 4
_STEPS = 3
_GAMMA = 0.01
_INV_GAMMA = 1.0 / _GAMMA


def _norm(lse):
    m = jnp.max(lse)
    return jnp.where(m > 1.0, lse / m, lse)


def _infer_body(x_ref, w_ref, it_ref, out_ref, body_sc, cl_sc, r_sc):
    jr = jax.lax.broadcasted_iota(jnp.int32, (_G, _G), 0)

    w = w_ref[...]
    wmx = jnp.max(w, axis=1, keepdims=True)
    we = jnp.exp(w - wmx)
    ws = we / jnp.sum(we, axis=1, keepdims=True)

    r_sc[...] = x_ref[...]

    for _ in range(_STEPS):
        R = r_sc[...]

        def clause_body(c, _):
            def s_pass1(s, mx):
                oh0 = (jr == it_ref[c, 0, s, :][None, :]).astype(jnp.float32)
                oh1 = (jr == it_ref[c, 1, s, :][None, :]).astype(jnp.float32)
                g0 = jax.lax.dot(R, oh0, preferred_element_type=jnp.float32)
                g1 = jax.lax.dot(R, oh1, preferred_element_type=jnp.float32)
                bd = g0 * g1
                body_sc[s] = bd
                return jnp.maximum(mx, bd)

            mx = jax.lax.fori_loop(
                0, _S, s_pass1, jnp.zeros((_B, _G), jnp.float32))

            def s_pass2(s, acc):
                return acc + jnp.exp((body_sc[s] - mx) * _INV_GAMMA)

            se = jax.lax.fori_loop(
                0, _S, s_pass2, jnp.zeros((_B, _G), jnp.float32))
            lse = _GAMMA * jnp.log(se) + mx
            cl_sc[c] = _norm(lse)
            return 0

        jax.lax.fori_loop(0, _C, clause_body, 0)

        hs = []
        for m in range(_M):
            h = cl_sc[0] * ws[m, 0]
            for c in range(1, _C):
                h = h + cl_sc[c] * ws[m, c]
            hs.append(h)
        hmx = jnp.maximum(jnp.maximum(hs[0], hs[1]),
                          jnp.maximum(hs[2], hs[3]))
        hse = sum(jnp.exp((h - hmx) * _INV_GAMMA) for h in hs)
        r = _norm(_GAMMA * jnp.log(hse) + hmx)

        pmx = jnp.maximum(R, r)
        pse = jnp.exp((R - pmx) * _INV_GAMMA) + jnp.exp((r - pmx) * _INV_GAMMA)
        r_sc[...] = _norm(_GAMMA * jnp.log(pse) + pmx)

    out_ref[...] = r_sc[...]


def kernel(x, W, I):
    it = jnp.transpose(I, (0, 3, 2, 1)).astype(jnp.int32)
    return pl.pallas_call(
        _infer_body,
        out_shape=jax.ShapeDtypeStruct((_B, _G), jnp.float32),
        scratch_shapes=[
            pltpu.VMEM((_S, _B, _G), jnp.float32),
            pltpu.VMEM((_C, _B, _G), jnp.float32),
            pltpu.VMEM((_B, _G), jnp.float32),
        ],
    )(x, W, it)


# speedup vs baseline: 19.9858x; 2.1142x over previous
import jax
import jax.numpy as jnp
from jax import lax
from jax.experimental import pallas as pl
from jax.experimental.pallas import tpu as pltpu
from jax.experimental.pallas import tpu_sc as plsc

_C, _G, _S, _L = 16, 1024, 32, 2
_B = 64
_M = 4
_STEPS = 3
_GAMMA = 0.01
_INV_GAMMA = 1.0 / _GAMMA
_CG = _C * _G
_NTILES = 32
_PER_TILE = _CG // _NTILES
_CHUNK = 128
_NCHUNK = _PER_TILE // _CHUNK
_NBG = _B // 16


def _sc_stage(xt10, idx):
    info = plsc.get_sparse_core_info()
    nc = info.num_cores
    mesh = plsc.VectorSubcoreMesh(core_axis_name="c", subcore_axis_name="s")

    idx_words = _PER_TILE * _L * _S

    def body(x_hbm, idx_hbm, mx_hbm, se_hbm, x_v, idx_v, mx_st, se_st):
        wid = lax.axis_index("s") * nc + lax.axis_index("c")
        pltpu.sync_copy(x_hbm, x_v)
        pltpu.sync_copy(idx_hbm.at[wid], idx_v)
        bvecs = [lax.iota(jnp.int32, 16) + (16 * g) for g in range(_NBG)]

        for chunk in range(_NCHUNK):
            def cg_body(j, carry):
                pbase = (chunk * _CHUNK + j) * (_L * _S)
                m = [jnp.zeros((16,), jnp.float32) for _ in range(_NBG)]
                sm = [jnp.zeros((16,), jnp.float32) for _ in range(_NBG)]
                iv0 = [idx_v[pl.ds(pbase + h * 16, 16)] * 64
                       for h in range(2)]
                iv1 = [idx_v[pl.ds(pbase + 32 + h * 16, 16)] * 64
                       for h in range(2)]
                for s in range(_S):
                    h, k = divmod(s, 16)
                    r0 = jnp.full((16,), iv0[h][k], jnp.int32)
                    r1 = jnp.full((16,), iv1[h][k], jnp.int32)
                    for g in range(_NBG):
                        v0 = plsc.load_gather(x_v, [r0 + bvecs[g]])
                        v1 = plsc.load_gather(x_v, [r1 + bvecs[g]])
                        v = v0 * v1
                        mn = jnp.maximum(m[g], v)
                        sm[g] = sm[g] * jnp.exp(m[g] - mn) + jnp.exp(v - mn)
                        m[g] = mn
                for g in range(_NBG):
                    mx_st[pl.ds(j * _B + g * 16, 16)] = m[g]
                    se_st[pl.ds(j * _B + g * 16, 16)] = sm[g]
                return carry

            lax.fori_loop(0, _CHUNK, cg_body, 0)
            out_slice = pl.ds((wid * _PER_TILE + chunk * _CHUNK) * _B,
                              _CHUNK * _B)
            pltpu.sync_copy(mx_st, mx_hbm.at[out_slice])
            pltpu.sync_copy(se_st, se_hbm.at[out_slice])

    f = pl.kernel(
        body,
        out_type=(jax.ShapeDtypeStruct((_CG * _B,), jnp.float32),
                  jax.ShapeDtypeStruct((_CG * _B,), jnp.float32)),
        mesh=mesh,
        compiler_params=pltpu.CompilerParams(needs_layout_passes=False),
        scratch_types=[
            pltpu.VMEM((_G * _B,), jnp.float32),
            pltpu.VMEM((idx_words,), jnp.int32),
            pltpu.VMEM((_CHUNK * _B,), jnp.float32),
            pltpu.VMEM((_CHUNK * _B,), jnp.float32),
        ],
    )
    return f(xt10.reshape(_G * _B), idx.reshape(_NTILES, idx_words))


def _norm(lse):
    m = jnp.max(lse)
    return jnp.where(m > 1.0, lse / m, lse)


def _tc_combine(mx, se, W, rt):
    def body(mx_ref, se_ref, w_ref, rt_ref, out_ref, out10_ref):
        w = w_ref[...]
        wmx = jnp.max(w, axis=1, keepdims=True)
        we = jnp.exp(w - wmx)
        ws = we / jnp.sum(we, axis=1, keepdims=True)

        hs = [jnp.zeros((_G, _B), jnp.float32) for _ in range(_M)]
        for c in range(_C):
            lse = _GAMMA * jnp.log(se_ref[c]) + _GAMMA * mx_ref[c]
            cl = _norm(lse)
            for m in range(_M):
                hs[m] = hs[m] + cl * ws[m, c]
        hmx = jnp.maximum(jnp.maximum(hs[0], hs[1]),
                          jnp.maximum(hs[2], hs[3]))
        hse = sum(jnp.exp((h - hmx) * _INV_GAMMA) for h in hs)
        r = _norm(_GAMMA * jnp.log(hse) + hmx)

        R = rt_ref[...]
        pmx = jnp.maximum(R, r)
        pse = jnp.exp((R - pmx) * _INV_GAMMA) + jnp.exp((r - pmx) * _INV_GAMMA)
        rn = _norm(_GAMMA * jnp.log(pse) + pmx)
        out_ref[...] = rn
        out10_ref[...] = rn * 10.0

    return pl.pallas_call(
        body,
        out_shape=(jax.ShapeDtypeStruct((_G, _B), jnp.float32),
                   jax.ShapeDtypeStruct((_G, _B), jnp.float32)),
    )(mx, se, W, rt)


def kernel(x, W, I):
    idx = jnp.transpose(I, (0, 1, 3, 2)).reshape(_CG, _L, _S).astype(jnp.int32)
    rt = jnp.transpose(x)
    rt10 = rt * 10.0
    for _ in range(_STEPS):
        mx, se = _sc_stage(rt10, idx)
        rt, rt10 = _tc_combine(mx.reshape(_C, _G, _B),
                               se.reshape(_C, _G, _B), W, rt)
    return jnp.transpose(rt)
